# Initial kernel scaffold; baseline (speedup 1.0000x reference)
import jax
import jax.numpy as jnp
from jax.experimental import pallas as pl


def kernel(inputs, hx, adj, w_fn, b_fn, w_g, b_g):
    raise NotImplementedError("write your pallas kernel here")


# speedup vs baseline: 1.4084x; 1.4084x over previous
import functools

import jax
import jax.numpy as jnp
from jax.experimental import pallas as pl

N = 1024
D = 64
U = 64
NM = 3
F = D + U


def _norm_kernel(at_ref, out_ref):
    a = at_ref[...]
    row = jax.lax.broadcasted_iota(jnp.int32, a.shape, 0)
    col = jax.lax.broadcasted_iota(jnp.int32, a.shape, 1)
    ap = a + jnp.where(row == col, 1.0, 0.0).astype(a.dtype)
    d = jnp.sum(ap, axis=0, keepdims=True)
    dinv = 1.0 / d
    dinv = jnp.where(jnp.isinf(dinv), 0.0, dinv)
    out_ref[...] = ap * dinv


def _mm2_kernel(a_ref, x_ref, h_ref, ox_ref, oh_ref):
    a = a_ref[...]
    ox_ref[...] = jnp.dot(a, x_ref[...], preferred_element_type=jnp.float32)
    oh_ref[...] = jnp.dot(a, h_ref[...], preferred_element_type=jnp.float32)


def _cheb2_kernel(a_ref, x_ref, h_ref, px_ref, ph_ref, ox_ref, oh_ref):
    a = a_ref[...]
    ox_ref[...] = 2.0 * jnp.dot(a, x_ref[...],
                                preferred_element_type=jnp.float32) - px_ref[...]
    oh_ref[...] = 2.0 * jnp.dot(a, h_ref[...],
                                preferred_element_type=jnp.float32) - ph_ref[...]


def _mm1_kernel(a_ref, x_ref, ox_ref):
    ox_ref[...] = jnp.dot(a_ref[...], x_ref[...],
                          preferred_element_type=jnp.float32)


def _cheb1_kernel(a_ref, x_ref, px_ref, ox_ref):
    ox_ref[...] = 2.0 * jnp.dot(a_ref[...], x_ref[...],
                                preferred_element_type=jnp.float32) - px_ref[...]


def _gate_kernel(x0_ref, h0_ref, p1_ref, h1_ref, p2_ref, h2_ref,
                 wf_ref, bf_ref, s0_ref, u_ref):
    h0 = h0_ref[...]
    cat = jnp.concatenate(
        [x0_ref[...], h0, p1_ref[...], h1_ref[...], p2_ref[...], h2_ref[...]],
        axis=1)
    g = jnp.dot(cat, wf_ref[...], preferred_element_type=jnp.float32)
    g = jax.nn.sigmoid(g + bf_ref[...])
    s0_ref[...] = g[:, :U] * h0
    u_ref[...] = g[:, U:]


def _final_kernel(x0_ref, s0_ref, p1_ref, s1_ref, p2_ref, s2_ref,
                  h0_ref, u_ref, wg_ref, bg_ref, out_ref):
    cat = jnp.concatenate(
        [x0_ref[...], s0_ref[...], p1_ref[...], s1_ref[...],
         p2_ref[...], s2_ref[...]],
        axis=1)
    c = jnp.dot(cat, wg_ref[...], preferred_element_type=jnp.float32)
    c = jnp.tanh(c + bg_ref[...])
    u = u_ref[...]
    out_ref[...] = u * h0_ref[...] + (1.0 - u) * c


def _full(shape):
    return pl.BlockSpec(shape, lambda j: (0, 0))


def _colblk(bn):
    return pl.BlockSpec((N, bn), lambda j: (0, j))


def _mm2(a, x, h, bn=512):
    w = x.shape[1]
    f32 = jnp.float32
    return pl.pallas_call(
        _mm2_kernel,
        grid=(w // bn,),
        in_specs=[_full((N, N)), _colblk(bn), _colblk(bn)],
        out_specs=[_colblk(bn), _colblk(bn)],
        out_shape=[jax.ShapeDtypeStruct((N, w), f32)] * 2,
    )(a, x, h)


def _cheb2(a, x, h, px, ph, bn=512):
    w = x.shape[1]
    f32 = jnp.float32
    return pl.pallas_call(
        _cheb2_kernel,
        grid=(w // bn,),
        in_specs=[_full((N, N))] + [_colblk(bn)] * 4,
        out_specs=[_colblk(bn), _colblk(bn)],
        out_shape=[jax.ShapeDtypeStruct((N, w), f32)] * 2,
    )(a, x, h, px, ph)


def _mm1(a, x, bn=512):
    w = x.shape[1]
    return pl.pallas_call(
        _mm1_kernel,
        grid=(w // bn,),
        in_specs=[_full((N, N)), _colblk(bn)],
        out_specs=_colblk(bn),
        out_shape=jax.ShapeDtypeStruct((N, w), jnp.float32),
    )(a, x)


def _cheb1(a, x, px, bn=512):
    w = x.shape[1]
    return pl.pallas_call(
        _cheb1_kernel,
        grid=(w // bn,),
        in_specs=[_full((N, N)), _colblk(bn), _colblk(bn)],
        out_specs=_colblk(bn),
        out_shape=jax.ShapeDtypeStruct((N, w), jnp.float32),
    )(a, x, px)


def _rowblk(bm, wdt):
    return pl.BlockSpec((bm, wdt), lambda i: (i, 0))


def kernel(inputs, hx, adj, w_fn, b_fn, w_g, b_g):
    B = inputs.shape[0]
    f32 = jnp.float32

    X = inputs.reshape(B, N, D).transpose(1, 0, 2).reshape(N, B * D)
    H = hx.reshape(B, N, U).transpose(1, 0, 2).reshape(N, B * U)

    Wf = w_fn.reshape(F, NM, 2 * U).transpose(1, 0, 2).reshape(NM * F, 2 * U)
    Wg = w_g.reshape(F, NM, U).transpose(1, 0, 2).reshape(NM * F, U)
    bf = b_fn.reshape(1, 2 * U)
    bg = b_g.reshape(1, U)

    A = pl.pallas_call(
        _norm_kernel,
        in_specs=[_full((N, N))],
        out_specs=_full((N, N)),
        out_shape=jax.ShapeDtypeStruct((N, N), f32),
        grid=(1,),
    )(adj.T)

    P1, H1 = _mm2(A, X, H)
    P2, H2 = _cheb2(A, P1, H1, X, H)

    NB = N * B
    bm = 4096
    grid = (NB // bm,)
    S0, Ubuf = pl.pallas_call(
        _gate_kernel,
        grid=grid,
        in_specs=[_rowblk(bm, D)] * 6 + [
            pl.BlockSpec((NM * F, 2 * U), lambda i: (0, 0)),
            pl.BlockSpec((1, 2 * U), lambda i: (0, 0)),
        ],
        out_specs=[_rowblk(bm, U), _rowblk(bm, U)],
        out_shape=[jax.ShapeDtypeStruct((NB, U), f32)] * 2,
    )(X.reshape(NB, D), H.reshape(NB, U), P1.reshape(NB, D),
      H1.reshape(NB, U), P2.reshape(NB, D), H2.reshape(NB, U), Wf, bf)

    S0m = S0.reshape(N, B * U)
    S1 = _mm1(A, S0m)
    S2 = _cheb1(A, S1, S0m)

    new = pl.pallas_call(
        _final_kernel,
        grid=grid,
        in_specs=[_rowblk(bm, U)] * 8 + [
            pl.BlockSpec((NM * F, U), lambda i: (0, 0)),
            pl.BlockSpec((1, U), lambda i: (0, 0)),
        ],
        out_specs=_rowblk(bm, U),
        out_shape=jax.ShapeDtypeStruct((NB, U), f32),
    )(X.reshape(NB, D), S0, P1.reshape(NB, D), S1.reshape(NB, U),
      P2.reshape(NB, D), S2.reshape(NB, U), H.reshape(NB, U), Ubuf, Wg, bg)

    return new.reshape(N, B, U).transpose(1, 0, 2).reshape(B, N * U)


# speedup vs baseline: 3.2619x; 2.3160x over previous
import jax
import jax.numpy as jnp
from jax.experimental import pallas as pl

N = 1024
D = 64
U = 64
NM = 3
F = D + U
BN = 256


def _norm_kernel(at_ref, out_ref):
    a = at_ref[...]
    row = jax.lax.broadcasted_iota(jnp.int32, a.shape, 0)
    col = jax.lax.broadcasted_iota(jnp.int32, a.shape, 1)
    ap = a + jnp.where(row == col, 1.0, 0.0).astype(a.dtype)
    d = jnp.sum(ap, axis=0, keepdims=True)
    dinv = 1.0 / d
    dinv = jnp.where(jnp.isinf(dinv), 0.0, dinv)
    out_ref[...] = ap * dinv


def _cell_kernel(a_ref, x_ref, h_ref, wf_ref, bf_ref, wg_ref, bg_ref,
                 out_ref):
    a = a_ref[...]
    x = x_ref[...]
    h = h_ref[...]

    def dot(p, q):
        return jnp.dot(p, q, preferred_element_type=jnp.float32)

    p1 = dot(a, x)
    h1 = dot(a, h)
    p2 = 2.0 * dot(a, p1) - x
    h2 = 2.0 * dot(a, h1) - h

    wf = wf_ref[...]
    bf = bf_ref[...]
    nb = BN // U
    u_parts = []
    s0_parts = []
    for b in range(nb):
        sl = slice(b * U, (b + 1) * U)
        cat = jnp.concatenate(
            [x[:, sl], h[:, sl], p1[:, sl], h1[:, sl], p2[:, sl], h2[:, sl]],
            axis=1)
        g = jax.nn.sigmoid(dot(cat, wf) + bf)
        u_parts.append(g[:, U:])
        s0_parts.append(g[:, :U] * h[:, sl])
    s0 = jnp.concatenate(s0_parts, axis=1)

    s1 = dot(a, s0)
    s2 = 2.0 * dot(a, s1) - s0

    wg = wg_ref[...]
    bg = bg_ref[...]
    for b in range(nb):
        sl = slice(b * U, (b + 1) * U)
        cat = jnp.concatenate(
            [x[:, sl], s0[:, sl], p1[:, sl], s1[:, sl], p2[:, sl], s2[:, sl]],
            axis=1)
        c = jnp.tanh(dot(cat, wg) + bg)
        u = u_parts[b]
        out_ref[:, sl] = u * h[:, sl] + (1.0 - u) * c


def _full(shape):
    return pl.BlockSpec(shape, lambda j: (0, 0))


def kernel(inputs, hx, adj, w_fn, b_fn, w_g, b_g):
    B = inputs.shape[0]
    f32 = jnp.float32

    X = inputs.reshape(B, N, D).transpose(1, 0, 2).reshape(N, B * D)
    H = hx.reshape(B, N, U).transpose(1, 0, 2).reshape(N, B * U)

    Wf = w_fn.reshape(F, NM, 2 * U).transpose(1, 0, 2).reshape(NM * F, 2 * U)
    Wg = w_g.reshape(F, NM, U).transpose(1, 0, 2).reshape(NM * F, U)
    bf = b_fn.reshape(1, 2 * U)
    bg = b_g.reshape(1, U)

    A = pl.pallas_call(
        _norm_kernel,
        in_specs=[_full((N, N))],
        out_specs=_full((N, N)),
        out_shape=jax.ShapeDtypeStruct((N, N), f32),
        grid=(1,),
    )(adj.T)

    stripe = pl.BlockSpec((N, BN), lambda j: (0, j))
    new = pl.pallas_call(
        _cell_kernel,
        grid=(B * U // BN,),
        in_specs=[
            _full((N, N)), stripe, stripe,
            _full((NM * F, 2 * U)), _full((1, 2 * U)),
            _full((NM * F, U)), _full((1, U)),
        ],
        out_specs=stripe,
        out_shape=jax.ShapeDtypeStruct((N, B * U), f32),
    )(A, X, H, Wf, bf, Wg, bg)

    return new.reshape(N, B, U).transpose(1, 0, 2).reshape(B, N * U)


# speedup vs baseline: 3.5559x; 1.0901x over previous
import jax
import jax.numpy as jnp
from jax import lax
from jax.experimental import pallas as pl

N = 1024
D = 64
U = 64
NM = 3
F = D + U
P = 2


def _norm_kernel(adj_ref, out_ref):
    a = adj_ref[...]
    row = lax.broadcasted_iota(jnp.int32, a.shape, 0)
    col = lax.broadcasted_iota(jnp.int32, a.shape, 1)
    ap = a + jnp.where(row == col, 1.0, 0.0).astype(a.dtype)
    d = jnp.sum(ap, axis=1, keepdims=True)
    dinv = 1.0 / d
    dinv = jnp.where(jnp.isinf(dinv), 0.0, dinv)
    out_ref[...] = ap * dinv


def _dotT(a, x):
    return lax.dot_general(a, x, (((0,), (0,)), ((), ())),
                           preferred_element_type=jnp.float32)


def _dot(a, b):
    return jnp.dot(a, b, preferred_element_type=jnp.float32)


def _cell_kernel(a_ref, x_ref, h_ref, wf_ref, bf_ref, wg_ref, bg_ref,
                 out_ref):
    a = a_ref[...]
    x = x_ref[...]
    h = h_ref[...]

    parts = []
    for i in range(P):
        parts.append(x[i])
        parts.append(h[i])
    xh = jnp.concatenate(parts, axis=1)

    g1 = _dotT(a, xh)
    g2 = 2.0 * _dotT(a, g1) - xh

    wf = wf_ref[...]
    bf = bf_ref[...]
    u_list = []
    s0_list = []
    for i in range(P):
        sl = slice(i * F, (i + 1) * F)
        cat = jnp.concatenate([xh[:, sl], g1[:, sl], g2[:, sl]], axis=1)
        g = jax.nn.sigmoid(_dot(cat, wf) + bf)
        u_list.append(g[:, U:])
        s0_list.append(g[:, :U] * h[i])
    s0 = jnp.concatenate(s0_list, axis=1)

    s1 = _dotT(a, s0)
    s2 = 2.0 * _dotT(a, s1) - s0

    wg = wg_ref[...]
    bg = bg_ref[...]
    for i in range(P):
        fl = slice(i * F, i * F + U)
        ul = slice(i * U, (i + 1) * U)
        cat = jnp.concatenate(
            [x[i], s0[:, ul], g1[:, fl], s1[:, ul], g2[:, fl], s2[:, ul]],
            axis=1)
        c = jnp.tanh(_dot(cat, wg) + bg)
        u = u_list[i]
        out_ref[i] = u * h[i] + (1.0 - u) * c


def _full(shape):
    return pl.BlockSpec(shape, lambda j: (0,) * len(shape))


def kernel(inputs, hx, adj, w_fn, b_fn, w_g, b_g):
    B = inputs.shape[0]
    f32 = jnp.float32

    X = inputs.reshape(B, N, D)
    H = hx.reshape(B, N, U)

    Wf = w_fn.reshape(F, NM, 2 * U).transpose(1, 0, 2).reshape(NM * F, 2 * U)
    Wg = w_g.reshape(F, NM, U).transpose(1, 0, 2).reshape(NM * F, U)
    bf = b_fn.reshape(1, 2 * U)
    bg = b_g.reshape(1, U)

    A = pl.pallas_call(
        _norm_kernel,
        in_specs=[_full((N, N))],
        out_specs=_full((N, N)),
        out_shape=jax.ShapeDtypeStruct((N, N), f32),
        grid=(1,),
    )(adj)

    grp = lambda w: pl.BlockSpec((P, N, w), lambda j: (j, 0, 0))
    new = pl.pallas_call(
        _cell_kernel,
        grid=(B // P,),
        in_specs=[
            _full((N, N)), grp(D), grp(U),
            _full((NM * F, 2 * U)), _full((1, 2 * U)),
            _full((NM * F, U)), _full((1, U)),
        ],
        out_specs=grp(U),
        out_shape=jax.ShapeDtypeStruct((B, N, U), f32),
    )(A, X, H, Wf, bf, Wg, bg)

    return new.reshape(B, N * U)
